# baseline (speedup 1.0000x reference)
import functools

import jax
import jax.numpy as jnp
from jax import lax
from jax.experimental import pallas as pl
from jax.experimental.pallas import tpu as pltpu
from jax.experimental.pallas import tpu_sc as plsc

N_NODES = 10000
N_EDGES = 320000
D = 128

NC = 2
NS = 16
NW = NC * NS
EPT = N_EDGES // NW
CHUNK = 104
NCH = EPT // CHUNK
TAIL = EPT - NCH * CHUNK
NBUF = 2
NSTEP = (NCH - 3) // NBUF
R_ACC = 10240
RPT = R_ACC // NS


def _sc_segment_sum(x, edge_r, z2d, z1d, ones_h):
    mesh = plsc.VectorSubcoreMesh(core_axis_name="c", subcore_axis_name="s")

    @functools.partial(
        pl.kernel,
        out_type=[
            jax.ShapeDtypeStruct((NC, R_ACC, D), jnp.float32),
            jax.ShapeDtypeStruct((NC * R_ACC,), jnp.float32),
        ],
        mesh=mesh,
        scratch_types=[
            pltpu.VMEM((EPT,), jnp.int32),
            pltpu.VMEM((EPT,), jnp.int32),
            pltpu.VMEM((NBUF, CHUNK, D), jnp.float32),
            pltpu.VMEM((CHUNK,), jnp.float32),
            pltpu.VMEM_SHARED((R_ACC, D), jnp.float32),
            pltpu.VMEM_SHARED((R_ACC,), jnp.float32),
            pltpu.SemaphoreType.DMA((NBUF,)),
            pltpu.SemaphoreType.DMA((NBUF,)),
            pltpu.SemaphoreType.DMA((NBUF,)),
        ],
    )
    def k(x_hbm, edge_hbm, z2_hbm, z1_hbm, ones_hbm,
          sum_out, cnt_out, src_v, dst_v, rows_v, ones_v, acc_sh, cnt_sh,
          gsem, ssem, csem):
        c = lax.axis_index("c")
        s = lax.axis_index("s")
        wid = s * NC + c

        pltpu.sync_copy(z2_hbm, acc_sh.at[pl.ds(s * RPT, RPT)])
        pltpu.sync_copy(z1_hbm.at[pl.ds(s * RPT, RPT)],
                        cnt_sh.at[pl.ds(s * RPT, RPT)])
        pltpu.sync_copy(ones_hbm, ones_v)
        pltpu.sync_copy(edge_hbm.at[pl.ds(wid * EPT, EPT)], src_v)
        pltpu.sync_copy(edge_hbm.at[pl.ds(N_EDGES + wid * EPT, EPT)], dst_v)
        plsc.subcore_barrier()

        def sidx(i, n=CHUNK):
            return src_v.at[pl.ds(i * CHUNK, n)]

        def didx(i, n=CHUNK):
            return dst_v.at[pl.ds(i * CHUNK, n)]

        def fire_gather(i, b):
            pltpu.async_copy(x_hbm.at[sidx(i)], rows_v.at[b], gsem.at[b])

        def wait_gather(i, b):
            pltpu.make_async_copy(x_hbm.at[sidx(i)], rows_v.at[b],
                                  gsem.at[b]).wait()

        def fire_scatter(i, b):
            pltpu.async_copy(rows_v.at[b], acc_sh.at[didx(i)],
                             ssem.at[b], add=True)
            pltpu.async_copy(ones_v, cnt_sh.at[didx(i)],
                             csem.at[0], add=True)

        def wait_scatter(i, b):
            pltpu.make_async_copy(rows_v.at[b], acc_sh.at[didx(i)],
                                  ssem.at[b]).wait()

        for b in range(NBUF):
            fire_gather(b, b)

        def step(kk, carry):
            for b in range(NBUF):
                i = kk * NBUF + b
                wait_gather(i, b)
                fire_scatter(i, b)
                wait_scatter(i, b)
                fire_gather(i + NBUF, b)
            return carry

        lax.fori_loop(0, NSTEP, step, 0)
        for i in range(NSTEP * NBUF, NCH):
            b = i % NBUF
            wait_gather(i, b)
            fire_scatter(i, b)
            wait_scatter(i, b)
            if i + NBUF < NCH:
                fire_gather(i + NBUF, b)
        tb = NCH % NBUF
        pltpu.async_copy(x_hbm.at[sidx(NCH, TAIL)],
                         rows_v.at[tb, pl.ds(0, TAIL)], gsem.at[tb])
        pltpu.make_async_copy(x_hbm.at[sidx(NCH, TAIL)],
                              rows_v.at[tb, pl.ds(0, TAIL)],
                              gsem.at[tb]).wait()
        pltpu.sync_copy(rows_v.at[tb, pl.ds(0, TAIL)],
                        acc_sh.at[didx(NCH, TAIL)], add=True)
        pltpu.sync_copy(ones_v.at[pl.ds(0, TAIL)],
                        cnt_sh.at[didx(NCH, TAIL)], add=True)

        def drain(_, carry):
            pltpu.make_async_copy(ones_v, cnt_sh.at[didx(0)],
                                  csem.at[0]).wait()
            return carry

        lax.fori_loop(0, NCH, drain, 0)
        plsc.subcore_barrier()

        pltpu.sync_copy(acc_sh.at[pl.ds(s * RPT, RPT)],
                        sum_out.at[c, pl.ds(s * RPT, RPT)])
        pltpu.sync_copy(cnt_sh.at[pl.ds(s * RPT, RPT)],
                        cnt_out.at[pl.ds(c * R_ACC + s * RPT, RPT)])

    return k(x, edge_r, z2d, z1d, ones_h)


def _tc_finish_body(p_ref, c_ref, x_ref, wl_ref, bl_ref, wr_ref, o_ref):
    summed = p_ref[0] + p_ref[1]
    cnt16 = c_ref[0] + c_ref[1]
    cnt_t = cnt16.T
    cnt = jnp.concatenate(
        [cnt_t[:, r:r + 1] for r in range(cnt_t.shape[1])], axis=0)
    mean = summed / jnp.maximum(cnt, 1.0)
    out = lax.dot_general(mean, wl_ref[...], (((1,), (1,)), ((), ())),
                          preferred_element_type=jnp.float32)
    out = out + lax.dot_general(x_ref[...], wr_ref[...],
                                (((1,), (1,)), ((), ())),
                                preferred_element_type=jnp.float32)
    out = out + bl_ref[...]
    nrm = jnp.sqrt(jnp.sum(out * out, axis=-1, keepdims=True))
    o_ref[...] = out / jnp.maximum(nrm, 1e-12)


def _tc_finish(partial_sum, partial_cnt, x, W_l, b_l, W_r):
    RB = 2048
    grid = (pl.cdiv(N_NODES, RB),)
    return pl.pallas_call(
        _tc_finish_body,
        grid=grid,
        in_specs=[
            pl.BlockSpec((NC, RB, D), lambda i: (0, i, 0)),
            pl.BlockSpec((NC, RB // D, D), lambda i: (0, i, 0)),
            pl.BlockSpec((RB, D), lambda i: (i, 0)),
            pl.BlockSpec((D, D), lambda i: (0, 0)),
            pl.BlockSpec((1, D), lambda i: (0, 0)),
            pl.BlockSpec((D, D), lambda i: (0, 0)),
        ],
        out_specs=pl.BlockSpec((RB, D), lambda i: (i, 0)),
        out_shape=jax.ShapeDtypeStruct((N_NODES, D), jnp.float32),
    )(partial_sum, partial_cnt, x, W_l, b_l, W_r)


def kernel(x, edge_index, W_l, b_l, W_r):
    edge_r = edge_index.astype(jnp.int32).reshape(2 * N_EDGES)
    z2d = jnp.zeros((RPT, D), jnp.float32)
    z1d = jnp.zeros((R_ACC,), jnp.float32)
    ones_h = jnp.ones((CHUNK,), jnp.float32)

    partial_sum, partial_cnt = _sc_segment_sum(x, edge_r, z2d, z1d, ones_h)
    return _tc_finish(partial_sum, partial_cnt.reshape(NC, R_ACC // D, D), x,
                      W_l, b_l.reshape(1, D), W_r)


# speedup vs baseline: 1.0562x; 1.0562x over previous
import functools

import jax
import jax.numpy as jnp
from jax import lax
from jax.experimental import pallas as pl
from jax.experimental.pallas import tpu as pltpu
from jax.experimental.pallas import tpu_sc as plsc

N_NODES = 10000
N_EDGES = 320000
D = 128

NC = 2
NS = 16
NW = NC * NS
EPT = N_EDGES // NW
CHUNK = 72
NCH = EPT // CHUNK
TAIL = EPT - NCH * CHUNK
NBUF = 3
NSTEP = (NCH - 3) // NBUF
R_ACC = 10240
RPT = R_ACC // NS


def _sc_segment_sum(x, edge_r, z2d, z1d, ones_h):
    mesh = plsc.VectorSubcoreMesh(core_axis_name="c", subcore_axis_name="s")

    @functools.partial(
        pl.kernel,
        out_type=[
            jax.ShapeDtypeStruct((NC, R_ACC, D), jnp.float32),
            jax.ShapeDtypeStruct((NC * R_ACC,), jnp.float32),
        ],
        mesh=mesh,
        scratch_types=[
            pltpu.VMEM((EPT,), jnp.int32),
            pltpu.VMEM((EPT,), jnp.int32),
            pltpu.VMEM((NBUF, CHUNK, D), jnp.float32),
            pltpu.VMEM((CHUNK,), jnp.float32),
            pltpu.VMEM_SHARED((R_ACC, D), jnp.float32),
            pltpu.VMEM_SHARED((R_ACC,), jnp.float32),
            pltpu.SemaphoreType.DMA((NBUF,)),
            pltpu.SemaphoreType.DMA((NBUF,)),
            pltpu.SemaphoreType.DMA((NBUF,)),
        ],
    )
    def k(x_hbm, edge_hbm, z2_hbm, z1_hbm, ones_hbm,
          sum_out, cnt_out, src_v, dst_v, rows_v, ones_v, acc_sh, cnt_sh,
          gsem, ssem, csem):
        c = lax.axis_index("c")
        s = lax.axis_index("s")
        wid = s * NC + c

        pltpu.sync_copy(z2_hbm, acc_sh.at[pl.ds(s * RPT, RPT)])
        pltpu.sync_copy(z1_hbm.at[pl.ds(s * RPT, RPT)],
                        cnt_sh.at[pl.ds(s * RPT, RPT)])
        pltpu.sync_copy(ones_hbm, ones_v)
        pltpu.sync_copy(edge_hbm.at[pl.ds(wid * EPT, EPT)], src_v)
        pltpu.sync_copy(edge_hbm.at[pl.ds(N_EDGES + wid * EPT, EPT)], dst_v)
        plsc.subcore_barrier()

        def sidx(i, n=CHUNK):
            return src_v.at[pl.ds(i * CHUNK, n)]

        def didx(i, n=CHUNK):
            return dst_v.at[pl.ds(i * CHUNK, n)]

        def fire_gather(i, b):
            pltpu.async_copy(x_hbm.at[sidx(i)], rows_v.at[b], gsem.at[b])

        def wait_gather(i, b):
            pltpu.make_async_copy(x_hbm.at[sidx(i)], rows_v.at[b],
                                  gsem.at[b]).wait()

        def fire_scatter(i, b):
            pltpu.async_copy(rows_v.at[b], acc_sh.at[didx(i)],
                             ssem.at[b], add=True)
            pltpu.async_copy(ones_v, cnt_sh.at[didx(i)],
                             csem.at[0], add=True)

        def wait_scatter(i, b):
            pltpu.make_async_copy(rows_v.at[b], acc_sh.at[didx(i)],
                                  ssem.at[b]).wait()

        for b in range(NBUF):
            fire_gather(b, b)
        wait_gather(0, 0)
        fire_scatter(0, 0)

        def step(kk, carry):
            for b in range(NBUF):
                i = kk * NBUF + b + 1
                wait_gather(i, (b + 1) % NBUF)
                fire_scatter(i, (b + 1) % NBUF)
                wait_scatter(i - 1, b)
                fire_gather(i + NBUF - 1, b)
            return carry

        lax.fori_loop(0, NSTEP, step, 0)
        for i in range(NSTEP * NBUF + 1, NCH):
            wait_gather(i, i % NBUF)
            fire_scatter(i, i % NBUF)
            wait_scatter(i - 1, (i - 1) % NBUF)
        wait_scatter(NCH - 1, (NCH - 1) % NBUF)
        tb = NCH % NBUF
        pltpu.async_copy(x_hbm.at[sidx(NCH, TAIL)],
                         rows_v.at[tb, pl.ds(0, TAIL)], gsem.at[tb])
        pltpu.make_async_copy(x_hbm.at[sidx(NCH, TAIL)],
                              rows_v.at[tb, pl.ds(0, TAIL)],
                              gsem.at[tb]).wait()
        pltpu.sync_copy(rows_v.at[tb, pl.ds(0, TAIL)],
                        acc_sh.at[didx(NCH, TAIL)], add=True)
        pltpu.sync_copy(ones_v.at[pl.ds(0, TAIL)],
                        cnt_sh.at[didx(NCH, TAIL)], add=True)

        def drain(_, carry):
            pltpu.make_async_copy(ones_v, cnt_sh.at[didx(0)],
                                  csem.at[0]).wait()
            return carry

        lax.fori_loop(0, NCH, drain, 0)
        plsc.subcore_barrier()

        pltpu.sync_copy(acc_sh.at[pl.ds(s * RPT, RPT)],
                        sum_out.at[c, pl.ds(s * RPT, RPT)])
        pltpu.sync_copy(cnt_sh.at[pl.ds(s * RPT, RPT)],
                        cnt_out.at[pl.ds(c * R_ACC + s * RPT, RPT)])

    return k(x, edge_r, z2d, z1d, ones_h)


def _tc_finish_body(p_ref, c_ref, x_ref, wl_ref, bl_ref, wr_ref, o_ref):
    summed = p_ref[0] + p_ref[1]
    cnt16 = c_ref[0] + c_ref[1]
    cnt_t = cnt16.T
    cnt = jnp.concatenate(
        [cnt_t[:, r:r + 1] for r in range(cnt_t.shape[1])], axis=0)
    mean = summed / jnp.maximum(cnt, 1.0)
    out = lax.dot_general(mean, wl_ref[...], (((1,), (1,)), ((), ())),
                          preferred_element_type=jnp.float32)
    out = out + lax.dot_general(x_ref[...], wr_ref[...],
                                (((1,), (1,)), ((), ())),
                                preferred_element_type=jnp.float32)
    out = out + bl_ref[...]
    nrm = jnp.sqrt(jnp.sum(out * out, axis=-1, keepdims=True))
    o_ref[...] = out / jnp.maximum(nrm, 1e-12)


def _tc_finish(partial_sum, partial_cnt, x, W_l, b_l, W_r):
    RB = 2048
    grid = (pl.cdiv(N_NODES, RB),)
    return pl.pallas_call(
        _tc_finish_body,
        grid=grid,
        in_specs=[
            pl.BlockSpec((NC, RB, D), lambda i: (0, i, 0)),
            pl.BlockSpec((NC, RB // D, D), lambda i: (0, i, 0)),
            pl.BlockSpec((RB, D), lambda i: (i, 0)),
            pl.BlockSpec((D, D), lambda i: (0, 0)),
            pl.BlockSpec((1, D), lambda i: (0, 0)),
            pl.BlockSpec((D, D), lambda i: (0, 0)),
        ],
        out_specs=pl.BlockSpec((RB, D), lambda i: (i, 0)),
        out_shape=jax.ShapeDtypeStruct((N_NODES, D), jnp.float32),
    )(partial_sum, partial_cnt, x, W_l, b_l, W_r)


def kernel(x, edge_index, W_l, b_l, W_r):
    edge_r = edge_index.astype(jnp.int32).reshape(2 * N_EDGES)
    z2d = jnp.zeros((RPT, D), jnp.float32)
    z1d = jnp.zeros((R_ACC,), jnp.float32)
    ones_h = jnp.ones((CHUNK,), jnp.float32)

    partial_sum, partial_cnt = _sc_segment_sum(x, edge_r, z2d, z1d, ones_h)
    return _tc_finish(partial_sum, partial_cnt.reshape(NC, R_ACC // D, D), x,
                      W_l, b_l.reshape(1, D), W_r)
